# baseline (speedup 1.0000x reference)
import functools

import jax
import jax.numpy as jnp
from jax import lax
from jax.experimental import pallas as pl
from jax.experimental.pallas import tpu as pltpu
from jax.experimental.pallas import tpu_sc as plsc

N, K, G, F = 10000, 32, 64, 128
NC, NS = 2, 16
NW = NC * NS
NBL = 256
CHUNKS = ((2816, 128), (3072, 128), (3584, 128), (528, 48))


def _sc_gather(feat2d, idx, n0, nc, ch):
    mesh = plsc.VectorSubcoreMesh(
        core_axis_name="c", subcore_axis_name="s", num_cores=NC, num_subcores=NS
    )
    nch = nc // ch

    @functools.partial(
        pl.kernel,
        out_type=jax.ShapeDtypeStruct((K * nc, F), jnp.float32),
        mesh=mesh,
        scratch_types=[
            pltpu.VMEM((nc,), jnp.int32),
            pltpu.VMEM((2, ch, F), jnp.float32),
            pltpu.VMEM_SHARED((N, F), jnp.float32),
            pltpu.SemaphoreType.DMA,
            pltpu.SemaphoreType.DMA,
        ],
    )
    def gather_k(feat_hbm, idx_hbm, out_hbm, idx_v, rows_v, feat_sh, gsem, osem):
        wid = lax.axis_index("s") * NC + lax.axis_index("c")
        sid = lax.axis_index("s")
        obase = wid * nc
        pltpu.sync_copy(
            feat_hbm.at[pl.ds(sid * 624, 624)],
            feat_sh.at[pl.ds(sid * 624, 624)],
        )

        @pl.when(sid == 0)
        def _():
            pltpu.sync_copy(
                feat_hbm.at[pl.ds(9984, N - 9984)],
                feat_sh.at[pl.ds(9984, N - 9984)],
            )
        pltpu.sync_copy(idx_hbm.at[pl.ds(wid * N + n0, nc)], idx_v)
        plsc.subcore_barrier()

        def start_gather(c, slot):
            pltpu.async_copy(
                feat_sh.at[idx_v.at[pl.ds(c * ch, ch)]], rows_v.at[slot], gsem
            )

        def wait_gather(c, slot):
            pltpu.make_async_copy(
                feat_sh.at[idx_v.at[pl.ds(c * ch, ch)]], rows_v.at[slot], gsem
            ).wait()

        def start_ocopy(c, slot):
            pltpu.async_copy(
                rows_v.at[slot], out_hbm.at[pl.ds(obase + c * ch, ch)], osem
            )

        def wait_ocopy(c, slot):
            pltpu.make_async_copy(
                rows_v.at[slot], out_hbm.at[pl.ds(obase + c * ch, ch)], osem
            ).wait()

        start_gather(0, 0)

        def body(c, _):
            slot = lax.rem(c, 2)
            nslot = lax.rem(c + 1, 2)

            @pl.when(jnp.logical_and(c + 1 < nch, c >= 1))
            def _():
                wait_ocopy(c - 1, nslot)

            @pl.when(c + 1 < nch)
            def _():
                start_gather(c + 1, nslot)

            wait_gather(c, slot)
            start_ocopy(c, slot)
            return 0

        lax.fori_loop(0, nch, body, 0)
        wait_ocopy(nch - 2, lax.rem(nch - 2, 2))
        wait_ocopy(nch - 1, lax.rem(nch - 1, 2))

    return gather_k(feat2d, idx)


def _tc_conv(rbf_t, nf3, w1t, b1, w2t, b2, out_acc, blk0, nc):
    grid = pl.cdiv(nc, NBL)

    def body(rbf_ref, nf_ref, w1_ref, b1_ref, w2_ref, b2_ref, _acc_ref, out_ref):
        r = rbf_ref[0]
        h = jnp.tanh(
            lax.dot_general(
                r, w1_ref[...], (((1,), (0,)), ((), ())),
                preferred_element_type=jnp.float32,
            )
            + b1_ref[...]
        )
        filt = (
            lax.dot_general(
                h, w2_ref[...], (((2,), (0,)), ((), ())),
                preferred_element_type=jnp.float32,
            )
            + b2_ref[...]
        )
        prod = filt * nf_ref[...]
        out_ref[...] = prod.sum(axis=0)

    return pl.pallas_call(
        body,
        grid=(grid,),
        in_specs=[
            pl.BlockSpec((1, K, G, NBL), lambda i: (0, 0, 0, i + blk0)),
            pl.BlockSpec((K, NBL, F), lambda i: (0, i, 0)),
            pl.BlockSpec((G, F), lambda i: (0, 0)),
            pl.BlockSpec((1, F), lambda i: (0, 0)),
            pl.BlockSpec((F, F), lambda i: (0, 0)),
            pl.BlockSpec((1, F), lambda i: (0, 0)),
            pl.BlockSpec(memory_space=pltpu.MemorySpace.HBM),
        ],
        out_specs=pl.BlockSpec((NBL, F), lambda i: (i + blk0, 0)),
        out_shape=jax.ShapeDtypeStruct((N, F), jnp.float32),
        input_output_aliases={6: 0},
        compiler_params=pltpu.CompilerParams(
            dimension_semantics=("arbitrary",)
        ),
    )(rbf_t, nf3, w1t, b1, w2t, b2, out_acc)


def kernel(features, rbf_expansion, neighbor_list, W1, b1, W2, b2):
    feat2d = features.reshape(N, F)
    idx = neighbor_list.transpose(0, 2, 1).reshape(K * N)
    rbf_t = rbf_expansion.transpose(0, 2, 3, 1)
    w1t, w2t = W1.T, W2.T
    b1r, b2r = b1.reshape(1, F), b2.reshape(1, F)

    out = jnp.zeros((N, F), jnp.float32)
    n0 = 0
    for nc, ch in CHUNKS:
        nf3 = _sc_gather(feat2d, idx, n0, nc, ch).reshape(K, nc, F)
        out = _tc_conv(rbf_t, nf3, w1t, b1r, w2t, b2r, out, n0 // NBL, nc)
        n0 += nc
    return out.reshape(1, N, F)


# speedup vs baseline: 1.0023x; 1.0023x over previous
import functools

import jax
import jax.numpy as jnp
from jax import lax
from jax.experimental import pallas as pl
from jax.experimental.pallas import tpu as pltpu
from jax.experimental.pallas import tpu_sc as plsc

N, K, G, F = 10000, 32, 64, 128
NC, NS = 2, 16
NW = NC * NS
NBL = 256
CHUNKS = ((2816, 128), (2816, 128), (3072, 128), (1296, 48))


def _sc_gather(feat2d, idx, n0, nc, ch):
    mesh = plsc.VectorSubcoreMesh(
        core_axis_name="c", subcore_axis_name="s", num_cores=NC, num_subcores=NS
    )
    nch = nc // ch

    @functools.partial(
        pl.kernel,
        out_type=jax.ShapeDtypeStruct((K * nc, F), jnp.float32),
        mesh=mesh,
        scratch_types=[
            pltpu.VMEM((nc,), jnp.int32),
            pltpu.VMEM((2, ch, F), jnp.float32),
            pltpu.VMEM_SHARED((N, F), jnp.float32),
            pltpu.SemaphoreType.DMA,
            pltpu.SemaphoreType.DMA,
        ],
    )
    def gather_k(feat_hbm, idx_hbm, out_hbm, idx_v, rows_v, feat_sh, gsem, osem):
        wid = lax.axis_index("s") * NC + lax.axis_index("c")
        sid = lax.axis_index("s")
        obase = wid * nc
        pltpu.sync_copy(
            feat_hbm.at[pl.ds(sid * 624, 624)],
            feat_sh.at[pl.ds(sid * 624, 624)],
        )

        @pl.when(sid == 0)
        def _():
            pltpu.sync_copy(
                feat_hbm.at[pl.ds(9984, N - 9984)],
                feat_sh.at[pl.ds(9984, N - 9984)],
            )
        pltpu.sync_copy(idx_hbm.at[pl.ds(wid * N + n0, nc)], idx_v)
        plsc.subcore_barrier()

        def start_gather(c, slot):
            pltpu.async_copy(
                feat_sh.at[idx_v.at[pl.ds(c * ch, ch)]], rows_v.at[slot], gsem
            )

        def wait_gather(c, slot):
            pltpu.make_async_copy(
                feat_sh.at[idx_v.at[pl.ds(c * ch, ch)]], rows_v.at[slot], gsem
            ).wait()

        def start_ocopy(c, slot):
            pltpu.async_copy(
                rows_v.at[slot], out_hbm.at[pl.ds(obase + c * ch, ch)], osem
            )

        def wait_ocopy(c, slot):
            pltpu.make_async_copy(
                rows_v.at[slot], out_hbm.at[pl.ds(obase + c * ch, ch)], osem
            ).wait()

        start_gather(0, 0)

        def body(c, _):
            slot = lax.rem(c, 2)
            nslot = lax.rem(c + 1, 2)

            @pl.when(jnp.logical_and(c + 1 < nch, c >= 1))
            def _():
                wait_ocopy(c - 1, nslot)

            @pl.when(c + 1 < nch)
            def _():
                start_gather(c + 1, nslot)

            wait_gather(c, slot)
            start_ocopy(c, slot)
            return 0

        lax.fori_loop(0, nch, body, 0)
        wait_ocopy(nch - 2, lax.rem(nch - 2, 2))
        wait_ocopy(nch - 1, lax.rem(nch - 1, 2))

    return gather_k(feat2d, idx)


def _tc_conv(rbf_t, nf3, w1t, b1, w2t, b2, out_acc, blk0, nc):
    grid = pl.cdiv(nc, NBL)

    def body(rbf_ref, nf_ref, w1_ref, b1_ref, w2_ref, b2_ref, _acc_ref, out_ref):
        r = rbf_ref[0]
        h = jnp.tanh(
            lax.dot_general(
                r, w1_ref[...], (((1,), (0,)), ((), ())),
                preferred_element_type=jnp.float32,
            )
            + b1_ref[...]
        )
        filt = (
            lax.dot_general(
                h, w2_ref[...], (((2,), (0,)), ((), ())),
                preferred_element_type=jnp.float32,
            )
            + b2_ref[...]
        )
        prod = filt * nf_ref[...]
        out_ref[...] = prod.sum(axis=0)

    return pl.pallas_call(
        body,
        grid=(grid,),
        in_specs=[
            pl.BlockSpec((1, K, G, NBL), lambda i: (0, 0, 0, i + blk0)),
            pl.BlockSpec((K, NBL, F), lambda i: (0, i, 0)),
            pl.BlockSpec((G, F), lambda i: (0, 0)),
            pl.BlockSpec((1, F), lambda i: (0, 0)),
            pl.BlockSpec((F, F), lambda i: (0, 0)),
            pl.BlockSpec((1, F), lambda i: (0, 0)),
            pl.BlockSpec(memory_space=pltpu.MemorySpace.HBM),
        ],
        out_specs=pl.BlockSpec((NBL, F), lambda i: (i + blk0, 0)),
        out_shape=jax.ShapeDtypeStruct((N, F), jnp.float32),
        input_output_aliases={6: 0},
        compiler_params=pltpu.CompilerParams(
            dimension_semantics=("arbitrary",)
        ),
    )(rbf_t, nf3, w1t, b1, w2t, b2, out_acc)


def kernel(features, rbf_expansion, neighbor_list, W1, b1, W2, b2):
    feat2d = features.reshape(N, F)
    idx = neighbor_list.transpose(0, 2, 1).reshape(K * N)
    rbf_t = rbf_expansion.transpose(0, 2, 3, 1)
    w1t, w2t = W1.T, W2.T
    b1r, b2r = b1.reshape(1, F), b2.reshape(1, F)

    out = jnp.zeros((N, F), jnp.float32)
    n0 = 0
    for nc, ch in CHUNKS:
        nf3 = _sc_gather(feat2d, idx, n0, nc, ch).reshape(K, nc, F)
        out = _tc_conv(rbf_t, nf3, w1t, b1r, w2t, b2r, out, n0 // NBL, nc)
        n0 += nc
    return out.reshape(1, N, F)
